# baseline (speedup 1.0000x reference)
import functools

import jax
import jax.numpy as jnp
from jax import lax
from jax.experimental import pallas as pl
from jax.experimental.pallas import tpu as pltpu
from jax.experimental.pallas import tpu_sc as plsc

_BSZ = 256


def _k1_body(x_ref, lab_ref,
             encWT_ref, encb_ref, embWT_ref,
             ze_ref, lg_ref, idx_ref, y_ref,
             *, dim, k, kp):
    x = x_ref[...]
    ze = jnp.dot(x, encWT_ref[...], preferred_element_type=jnp.float32) + encb_ref[...]
    logits = jnp.dot(ze, embWT_ref[...], preferred_element_type=jnp.float32)

    kiota = lax.broadcasted_iota(jnp.int32, logits.shape, 1)
    lg_m = jnp.where(kiota < k, logits, jnp.float32(-1e30))
    m = jnp.max(lg_m, axis=1, keepdims=True)
    e = jnp.exp(lg_m - m)
    soft = e / jnp.sum(e, axis=1, keepdims=True)
    idxmin = jnp.min(jnp.where(lg_m >= m, kiota, kp), axis=1, keepdims=True)

    sel = lab_ref[...][:, 0:1]
    onehot = (kiota == sel).astype(jnp.float32)
    y = jnp.where(sel >= 0, onehot, soft)

    ze_ref[...] = ze[:, :dim]
    lg_ref[...] = logits[:, :k]
    idx_ref[...] = jnp.broadcast_to(idxmin, idx_ref.shape)
    y_ref[...] = y


def _k2_body(x_ref, eps_ref, y_ref,
             W1T_ref, b1_ref, W2T_ref, b2_ref,
             decWzT_ref, decWyT_ref, decb_ref,
             xt_ref, kl_ref,
             *, n_rows, ngrid, vd):
    i = pl.program_id(0)
    x = x_ref[...]
    h = jnp.maximum(jnp.dot(x, W1T_ref[...], preferred_element_type=jnp.float32)
                    + b1_ref[...], 0.0)
    mv = jnp.dot(h, W2T_ref[...], preferred_element_type=jnp.float32) + b2_ref[...]
    mu = mv[:, :vd]
    logvar = mv[:, vd:]
    var = jnp.exp(logvar)
    std = jnp.exp(0.5 * logvar)
    sampled = mu + std * eps_ref[...]

    xt = (jnp.dot(sampled, decWzT_ref[...], preferred_element_type=jnp.float32)
          + jnp.dot(y_ref[...], decWyT_ref[...], preferred_element_type=jnp.float32)
          + decb_ref[...])
    xt_ref[...] = xt

    part = 0.5 * jnp.sum(var + mu * mu - 1.0 - logvar)
    prev = jnp.where(i == 0, jnp.zeros((1, 1), jnp.float32), kl_ref[...])
    tot = prev + part
    kl_ref[...] = jnp.where(i == ngrid - 1, tot / (n_rows * ngrid), tot)


def _sc_gather(table, idx):
    b = idx.shape[0]
    d = table.shape[1]
    nw = 32
    rows_pw = b // nw
    ch = 128
    nch = rows_pw // ch
    mesh = plsc.VectorSubcoreMesh(core_axis_name="c", subcore_axis_name="s")

    @functools.partial(
        pl.kernel, mesh=mesh,
        out_type=jax.ShapeDtypeStruct((b, d), jnp.float32),
        scratch_types=[
            pltpu.VMEM((ch,), jnp.int32),
            pltpu.VMEM((ch, d), jnp.float32),
            pltpu.SemaphoreType.DMA,
        ],
    )
    def k(table_hbm, idx_hbm, out_hbm, idx_v, rows_v, sem):
        wid = lax.axis_index("s") * 2 + lax.axis_index("c")
        base = wid * rows_pw
        for c in range(nch):
            off = base + c * ch
            pltpu.sync_copy(idx_hbm.at[pl.ds(off, ch)], idx_v)
            pltpu.async_copy(table_hbm.at[idx_v], rows_v, sem).wait()
            pltpu.sync_copy(rows_v, out_hbm.at[pl.ds(off, ch)])

    return k(table, idx)


def kernel(x, known_mask, labels, enc_W, enc_b, emb_W,
           vae_enc_W1, vae_enc_b1, vae_enc_W2, vae_enc_b2,
           vae_dec_W, vae_dec_b, eps):
    f32 = jnp.float32
    B, IN = x.shape
    DIM = enc_W.shape[0]
    K = emb_W.shape[0]
    VD = vae_enc_b1.shape[0]
    DP, KP = 512, 128
    bsz = _BSZ
    ngrid = B // bsz

    encWT = jnp.zeros((IN, DP), f32).at[:, :DIM].set(enc_W.T)
    encb = jnp.zeros((1, DP), f32).at[0, :DIM].set(enc_b)
    embWT = jnp.zeros((DP, KP), f32).at[:DIM, :K].set(emb_W.T)
    W1T = vae_enc_W1.T
    b1 = vae_enc_b1[None, :]
    W2T = vae_enc_W2.T
    b2 = vae_enc_b2[None, :]
    decWzT = vae_dec_W[:, :VD].T
    decWyT = jnp.zeros((KP, IN), f32).at[:K, :].set(vae_dec_W[:, VD:].T)
    decb = vae_dec_b[None, :]
    labm = jnp.where(known_mask, labels.astype(jnp.int32), -1)
    lab8 = jnp.broadcast_to(labm[:, None], (B, 8))

    def row_spec(w):
        return pl.BlockSpec((bsz, w), lambda i: (i, 0))

    def full(s):
        return pl.BlockSpec(s, lambda i: (0,) * len(s))

    ze, lg, idx8, y = pl.pallas_call(
        functools.partial(_k1_body, dim=DIM, k=K, kp=KP),
        grid=(ngrid,),
        in_specs=[row_spec(IN), row_spec(8),
                  full((IN, DP)), full((1, DP)), full((DP, KP))],
        out_specs=[row_spec(DIM), row_spec(K), row_spec(8), row_spec(KP)],
        out_shape=[jax.ShapeDtypeStruct((B, DIM), f32),
                   jax.ShapeDtypeStruct((B, K), f32),
                   jax.ShapeDtypeStruct((B, 8), jnp.int32),
                   jax.ShapeDtypeStruct((B, KP), f32)],
        compiler_params=pltpu.CompilerParams(
            dimension_semantics=("arbitrary",)),
    )(x, lab8, encWT, encb, embWT)

    embp = jnp.zeros((K, DP), f32).at[:, :DIM].set(emb_W)
    zqp = _sc_gather(embp, idx8[:, 0])

    xt, klacc = pl.pallas_call(
        functools.partial(_k2_body, n_rows=bsz, ngrid=ngrid, vd=VD),
        grid=(ngrid,),
        in_specs=[row_spec(IN), row_spec(IN), row_spec(KP),
                  full((IN, VD)), full((1, VD)),
                  full((VD, 2 * VD)), full((1, 2 * VD)),
                  full((VD, IN)), full((KP, IN)), full((1, IN))],
        out_specs=[row_spec(IN), full((1, 1))],
        out_shape=[jax.ShapeDtypeStruct((B, IN), f32),
                   jax.ShapeDtypeStruct((1, 1), f32)],
        compiler_params=pltpu.CompilerParams(
            dimension_semantics=("arbitrary",)),
    )(x, eps, y, W1T, b1, W2T, b2, decWzT, decWyT, decb)

    return (xt, ze, zqp[:, :DIM], lg, klacc[0, 0])


# speedup vs baseline: 1.0711x; 1.0711x over previous
import functools

import jax
import jax.numpy as jnp
from jax import lax
from jax.experimental import pallas as pl
from jax.experimental.pallas import tpu as pltpu
from jax.experimental.pallas import tpu_sc as plsc

_BSZ = 256


def _fused_body(x_ref, lab_ref, eps_ref,
                encWT_ref, encb_ref, embWT_ref,
                W1T_ref, b1_ref, W2T_ref, b2_ref,
                decWzT_ref, decWyT_ref, decb_ref,
                xt_ref, ze_ref, lg_ref, idx_ref, kl_ref,
                *, n_rows, ngrid, dim, k, vd, kp):
    i = pl.program_id(0)
    x = x_ref[...]

    ze = jnp.dot(x, encWT_ref[...], preferred_element_type=jnp.float32) + encb_ref[...]
    logits = jnp.dot(ze, embWT_ref[...], preferred_element_type=jnp.float32)

    kiota = lax.broadcasted_iota(jnp.int32, logits.shape, 1)
    lg_m = jnp.where(kiota < k, logits, jnp.float32(-1e30))
    m = jnp.max(lg_m, axis=1, keepdims=True)
    e = jnp.exp(lg_m - m)
    soft = e / jnp.sum(e, axis=1, keepdims=True)
    idx_ref[...] = jnp.min(jnp.where(lg_m >= m, kiota, kp), axis=1)

    sel = lab_ref[...][:, 0:1]
    onehot = (kiota == sel).astype(jnp.float32)
    y = jnp.where(sel >= 0, onehot, soft)

    h = jnp.maximum(jnp.dot(x, W1T_ref[...], preferred_element_type=jnp.float32)
                    + b1_ref[...], 0.0)
    mv = jnp.dot(h, W2T_ref[...], preferred_element_type=jnp.float32) + b2_ref[...]
    mu = mv[:, :vd]
    logvar = mv[:, vd:]
    var = jnp.exp(logvar)
    std = jnp.exp(0.5 * logvar)
    sampled = mu + std * eps_ref[...]

    xt = (jnp.dot(sampled, decWzT_ref[...], preferred_element_type=jnp.float32)
          + jnp.dot(y, decWyT_ref[...], preferred_element_type=jnp.float32)
          + decb_ref[...])

    xt_ref[...] = xt
    ze_ref[...] = ze[:, :dim]
    lg_ref[...] = logits[:, :k]

    part = 0.5 * jnp.sum(var + mu * mu - 1.0 - logvar)
    prev = jnp.where(i == 0, jnp.zeros((1, 1), jnp.float32), kl_ref[...])
    tot = prev + part
    kl_ref[...] = jnp.where(i == ngrid - 1, tot / (n_rows * ngrid), tot)


def _sc_gather(table, idx):
    b = idx.shape[0]
    d = table.shape[1]
    nw = 32
    rows_pw = b // nw
    ch = 128
    nch = rows_pw // ch
    mesh = plsc.VectorSubcoreMesh(core_axis_name="c", subcore_axis_name="s")

    @functools.partial(
        pl.kernel, mesh=mesh,
        out_type=jax.ShapeDtypeStruct((b, d), jnp.float32),
        scratch_types=[
            pltpu.VMEM((ch,), jnp.int32),
            pltpu.VMEM((ch, d), jnp.float32),
            pltpu.SemaphoreType.DMA,
        ],
    )
    def k(table_hbm, idx_hbm, out_hbm, idx_v, rows_v, sem):
        wid = lax.axis_index("s") * 2 + lax.axis_index("c")
        base = wid * rows_pw
        for c in range(nch):
            off = base + c * ch
            pltpu.sync_copy(idx_hbm.at[pl.ds(off, ch)], idx_v)
            pltpu.async_copy(table_hbm.at[idx_v], rows_v, sem).wait()
            pltpu.sync_copy(rows_v, out_hbm.at[pl.ds(off, ch)])

    return k(table, idx)


def kernel(x, known_mask, labels, enc_W, enc_b, emb_W,
           vae_enc_W1, vae_enc_b1, vae_enc_W2, vae_enc_b2,
           vae_dec_W, vae_dec_b, eps):
    f32 = jnp.float32
    B, IN = x.shape
    DIM = enc_W.shape[0]
    K = emb_W.shape[0]
    VD = vae_enc_b1.shape[0]
    DP, KP = 512, 128
    bsz = _BSZ
    ngrid = B // bsz

    encWT = jnp.zeros((IN, DP), f32).at[:, :DIM].set(enc_W.T)
    encb = jnp.zeros((1, DP), f32).at[0, :DIM].set(enc_b)
    embWT = jnp.zeros((DP, KP), f32).at[:DIM, :K].set(emb_W.T)
    W1T = vae_enc_W1.T
    b1 = vae_enc_b1[None, :]
    W2T = vae_enc_W2.T
    b2 = vae_enc_b2[None, :]
    decWzT = vae_dec_W[:, :VD].T
    decWyT = jnp.zeros((KP, IN), f32).at[:K, :].set(vae_dec_W[:, VD:].T)
    decb = vae_dec_b[None, :]
    labm = jnp.where(known_mask, labels.astype(jnp.int32), -1)
    lab8 = jnp.broadcast_to(labm[:, None], (B, 8))

    def row_spec(w):
        return pl.BlockSpec((bsz, w), lambda i: (i, 0))

    def full(s):
        return pl.BlockSpec(s, lambda i: (0,) * len(s))

    body = functools.partial(_fused_body, n_rows=bsz, ngrid=ngrid,
                             dim=DIM, k=K, vd=VD, kp=KP)
    xt, ze, lg, idx1, klacc = pl.pallas_call(
        body,
        grid=(ngrid,),
        in_specs=[row_spec(IN), row_spec(8), row_spec(IN),
                  full((IN, DP)), full((1, DP)), full((DP, KP)),
                  full((IN, VD)), full((1, VD)),
                  full((VD, 2 * VD)), full((1, 2 * VD)),
                  full((VD, IN)), full((KP, IN)), full((1, IN))],
        out_specs=[row_spec(IN), row_spec(DIM), row_spec(K),
                   pl.BlockSpec((bsz,), lambda i: (i,)),
                   full((1, 1))],
        out_shape=[jax.ShapeDtypeStruct((B, IN), f32),
                   jax.ShapeDtypeStruct((B, DIM), f32),
                   jax.ShapeDtypeStruct((B, K), f32),
                   jax.ShapeDtypeStruct((B,), jnp.int32),
                   jax.ShapeDtypeStruct((1, 1), f32)],
        compiler_params=pltpu.CompilerParams(
            dimension_semantics=("arbitrary",)),
    )(x, lab8, eps, encWT, encb, embWT, W1T, b1, W2T, b2,
      decWzT, decWyT, decb)

    embp = jnp.zeros((K, DP), f32).at[:, :DIM].set(emb_W)
    z_q = _sc_gather(embp, idx1)[:, :DIM]
    return (xt, ze, z_q, lg, klacc[0, 0])


# speedup vs baseline: 1.1126x; 1.0388x over previous
import functools

import jax
import jax.numpy as jnp
from jax import lax
from jax.experimental import pallas as pl
from jax.experimental.pallas import tpu as pltpu
from jax.experimental.pallas import tpu_sc as plsc

_BSZ = 256


def _fused_body(x_ref, lab_ref, eps_ref,
                encWT_ref, encb_ref, embWT_ref,
                W1T_ref, b1_ref, W2T_ref, b2_ref,
                decWzT_ref, decWyT_ref, decb_ref,
                xt_ref, ze_ref, lg_ref, idx_ref, kl_ref,
                *, n_rows, ngrid, dim, k, vd, kp):
    i = pl.program_id(0)
    x = x_ref[...]

    ze = jnp.dot(x, encWT_ref[...], preferred_element_type=jnp.float32) + encb_ref[...]
    logits = jnp.dot(ze, embWT_ref[...], preferred_element_type=jnp.float32)

    kiota = lax.broadcasted_iota(jnp.int32, logits.shape, 1)
    lg_m = jnp.where(kiota < k, logits, jnp.float32(-1e30))
    m = jnp.max(lg_m, axis=1, keepdims=True)
    e = jnp.exp(lg_m - m)
    soft = e / jnp.sum(e, axis=1, keepdims=True)
    idx_ref[...] = jnp.min(jnp.where(lg_m >= m, kiota, kp), axis=1)

    sel = lab_ref[...][:, 0:1]
    onehot = (kiota == sel).astype(jnp.float32)
    y = jnp.where(sel >= 0, onehot, soft)

    bf16 = jnp.bfloat16
    h = jnp.maximum(jnp.dot(x.astype(bf16), W1T_ref[...],
                            preferred_element_type=jnp.float32)
                    + b1_ref[...], 0.0)
    mv = jnp.dot(h.astype(bf16), W2T_ref[...],
                 preferred_element_type=jnp.float32) + b2_ref[...]
    mu = mv[:, :vd]
    logvar = mv[:, vd:]
    var = jnp.exp(logvar)
    std = jnp.exp(0.5 * logvar)
    sampled = mu + std * eps_ref[...]

    xt = (jnp.dot(sampled.astype(bf16), decWzT_ref[...],
                  preferred_element_type=jnp.float32)
          + jnp.dot(y.astype(bf16), decWyT_ref[...],
                    preferred_element_type=jnp.float32)
          + decb_ref[...])

    xt_ref[...] = xt
    ze_ref[...] = ze[:, :dim]
    lg_ref[...] = logits[:, :k]

    part = 0.5 * jnp.sum(var + mu * mu - 1.0 - logvar)
    prev = jnp.where(i == 0, jnp.zeros((1, 1), jnp.float32), kl_ref[...])
    tot = prev + part
    kl_ref[...] = jnp.where(i == ngrid - 1, tot / (n_rows * ngrid), tot)


def _sc_gather(table, idx):
    b = idx.shape[0]
    d = table.shape[1]
    nw = 32
    rows_pw = b // nw
    ch = 128
    nch = rows_pw // ch
    mesh = plsc.VectorSubcoreMesh(core_axis_name="c", subcore_axis_name="s")

    @functools.partial(
        pl.kernel, mesh=mesh,
        out_type=jax.ShapeDtypeStruct((b, d), jnp.float32),
        scratch_types=[
            pltpu.VMEM((ch,), jnp.int32),
            pltpu.VMEM((ch, d), jnp.float32),
            pltpu.SemaphoreType.DMA,
        ],
    )
    def k(table_hbm, idx_hbm, out_hbm, idx_v, rows_v, sem):
        wid = lax.axis_index("s") * 2 + lax.axis_index("c")
        base = wid * rows_pw
        for c in range(nch):
            off = base + c * ch
            pltpu.sync_copy(idx_hbm.at[pl.ds(off, ch)], idx_v)
            pltpu.async_copy(table_hbm.at[idx_v], rows_v, sem).wait()
            pltpu.sync_copy(rows_v, out_hbm.at[pl.ds(off, ch)])

    return k(table, idx)


def kernel(x, known_mask, labels, enc_W, enc_b, emb_W,
           vae_enc_W1, vae_enc_b1, vae_enc_W2, vae_enc_b2,
           vae_dec_W, vae_dec_b, eps):
    f32 = jnp.float32
    B, IN = x.shape
    DIM = enc_W.shape[0]
    K = emb_W.shape[0]
    VD = vae_enc_b1.shape[0]
    DP, KP = 512, 128
    bsz = _BSZ
    ngrid = B // bsz

    encWT = jnp.zeros((IN, DP), f32).at[:, :DIM].set(enc_W.T)
    encb = jnp.zeros((1, DP), f32).at[0, :DIM].set(enc_b)
    embWT = jnp.zeros((DP, KP), f32).at[:DIM, :K].set(emb_W.T)
    bf16 = jnp.bfloat16
    W1T = vae_enc_W1.T.astype(bf16)
    b1 = vae_enc_b1[None, :]
    W2T = vae_enc_W2.T.astype(bf16)
    b2 = vae_enc_b2[None, :]
    decWzT = vae_dec_W[:, :VD].T.astype(bf16)
    decWyT = jnp.zeros((KP, IN), bf16).at[:K, :].set(
        vae_dec_W[:, VD:].T.astype(bf16))
    decb = vae_dec_b[None, :]
    labm = jnp.where(known_mask, labels.astype(jnp.int32), -1)
    lab8 = jnp.broadcast_to(labm[:, None], (B, 8))

    def row_spec(w):
        return pl.BlockSpec((bsz, w), lambda i: (i, 0))

    def full(s):
        return pl.BlockSpec(s, lambda i: (0,) * len(s))

    body = functools.partial(_fused_body, n_rows=bsz, ngrid=ngrid,
                             dim=DIM, k=K, vd=VD, kp=KP)
    xt, ze, lg, idx1, klacc = pl.pallas_call(
        body,
        grid=(ngrid,),
        in_specs=[row_spec(IN), row_spec(8), row_spec(IN),
                  full((IN, DP)), full((1, DP)), full((DP, KP)),
                  full((IN, VD)), full((1, VD)),
                  full((VD, 2 * VD)), full((1, 2 * VD)),
                  full((VD, IN)), full((KP, IN)), full((1, IN))],
        out_specs=[row_spec(IN), row_spec(DIM), row_spec(K),
                   pl.BlockSpec((bsz,), lambda i: (i,)),
                   full((1, 1))],
        out_shape=[jax.ShapeDtypeStruct((B, IN), f32),
                   jax.ShapeDtypeStruct((B, DIM), f32),
                   jax.ShapeDtypeStruct((B, K), f32),
                   jax.ShapeDtypeStruct((B,), jnp.int32),
                   jax.ShapeDtypeStruct((1, 1), f32)],
        compiler_params=pltpu.CompilerParams(
            dimension_semantics=("arbitrary",)),
    )(x, lab8, eps, encWT, encb, embWT, W1T, b1, W2T, b2,
      decWzT, decWyT, decb)

    embp = jnp.zeros((K, DP), f32).at[:, :DIM].set(emb_W)
    z_q = _sc_gather(embp, idx1)[:, :DIM]
    return (xt, ze, z_q, lg, klacc[0, 0])


# speedup vs baseline: 1.1493x; 1.0329x over previous
import functools

import jax
import jax.numpy as jnp
from jax import lax
from jax.experimental import pallas as pl
from jax.experimental.pallas import tpu as pltpu
from jax.experimental.pallas import tpu_sc as plsc

_BSZ = 512


def _fused_body(x_ref, lab_ref, eps_ref,
                encWT_ref, encb_ref, embWT_ref,
                W1T_ref, b1_ref, W2T_ref, b2_ref,
                decWzT_ref, decWyT_ref, decb_ref,
                xt_ref, ze_ref, lg_ref, idx_ref, kl_ref,
                *, n_rows, ngrid, dim, k, vd, kp):
    i = pl.program_id(0)
    x = x_ref[...]

    ze = jnp.dot(x, encWT_ref[...], preferred_element_type=jnp.float32) + encb_ref[...]
    logits = jnp.dot(ze, embWT_ref[...], preferred_element_type=jnp.float32)

    kiota = lax.broadcasted_iota(jnp.int32, logits.shape, 1)
    lg_m = jnp.where(kiota < k, logits, jnp.float32(-1e30))
    m = jnp.max(lg_m, axis=1, keepdims=True)
    e = jnp.exp(lg_m - m)
    soft = e / jnp.sum(e, axis=1, keepdims=True)
    idx_ref[...] = jnp.min(jnp.where(lg_m >= m, kiota, kp), axis=1)

    sel = lab_ref[...][:, 0:1]
    onehot = (kiota == sel).astype(jnp.float32)
    y = jnp.where(sel >= 0, onehot, soft)

    bf16 = jnp.bfloat16
    h = jnp.maximum(jnp.dot(x.astype(bf16), W1T_ref[...],
                            preferred_element_type=jnp.float32)
                    + b1_ref[...], 0.0)
    mv = jnp.dot(h.astype(bf16), W2T_ref[...],
                 preferred_element_type=jnp.float32) + b2_ref[...]
    mu = mv[:, :vd]
    logvar = mv[:, vd:]
    var = jnp.exp(logvar)
    std = jnp.exp(0.5 * logvar)
    sampled = mu + std * eps_ref[...]

    xt = (jnp.dot(sampled.astype(bf16), decWzT_ref[...],
                  preferred_element_type=jnp.float32)
          + jnp.dot(y.astype(bf16), decWyT_ref[...],
                    preferred_element_type=jnp.float32)
          + decb_ref[...])

    xt_ref[...] = xt
    ze_ref[...] = ze[:, :dim]
    lg_ref[...] = logits[:, :k]

    part = 0.5 * jnp.sum(var + mu * mu - 1.0 - logvar)
    prev = jnp.where(i == 0, jnp.zeros((1, 1), jnp.float32), kl_ref[...])
    tot = prev + part
    kl_ref[...] = jnp.where(i == ngrid - 1, tot / (n_rows * ngrid), tot)


def _sc_gather(table, idx):
    b = idx.shape[0]
    d = table.shape[1]
    nw = 32
    rows_pw = b // nw
    ch = 128
    nch = rows_pw // ch
    mesh = plsc.VectorSubcoreMesh(core_axis_name="c", subcore_axis_name="s")

    @functools.partial(
        pl.kernel, mesh=mesh,
        out_type=jax.ShapeDtypeStruct((b, d), jnp.float32),
        scratch_types=[
            pltpu.VMEM((ch,), jnp.int32),
            pltpu.VMEM((ch, d), jnp.float32),
            pltpu.SemaphoreType.DMA,
        ],
    )
    def k(table_hbm, idx_hbm, out_hbm, idx_v, rows_v, sem):
        wid = lax.axis_index("s") * 2 + lax.axis_index("c")
        base = wid * rows_pw
        for c in range(nch):
            off = base + c * ch
            pltpu.sync_copy(idx_hbm.at[pl.ds(off, ch)], idx_v)
            pltpu.async_copy(table_hbm.at[idx_v], rows_v, sem).wait()
            pltpu.sync_copy(rows_v, out_hbm.at[pl.ds(off, ch)])

    return k(table, idx)


def kernel(x, known_mask, labels, enc_W, enc_b, emb_W,
           vae_enc_W1, vae_enc_b1, vae_enc_W2, vae_enc_b2,
           vae_dec_W, vae_dec_b, eps):
    f32 = jnp.float32
    B, IN = x.shape
    DIM = enc_W.shape[0]
    K = emb_W.shape[0]
    VD = vae_enc_b1.shape[0]
    DP, KP = 512, 128
    bsz = _BSZ
    ngrid = B // bsz

    encWT = jnp.zeros((IN, DP), f32).at[:, :DIM].set(enc_W.T)
    encb = jnp.zeros((1, DP), f32).at[0, :DIM].set(enc_b)
    embWT = jnp.zeros((DP, KP), f32).at[:DIM, :K].set(emb_W.T)
    bf16 = jnp.bfloat16
    W1T = vae_enc_W1.T.astype(bf16)
    b1 = vae_enc_b1[None, :]
    W2T = vae_enc_W2.T.astype(bf16)
    b2 = vae_enc_b2[None, :]
    decWzT = vae_dec_W[:, :VD].T.astype(bf16)
    decWyT = jnp.zeros((KP, IN), bf16).at[:K, :].set(
        vae_dec_W[:, VD:].T.astype(bf16))
    decb = vae_dec_b[None, :]
    labm = jnp.where(known_mask, labels.astype(jnp.int32), -1)
    lab8 = jnp.broadcast_to(labm[:, None], (B, 8))

    def row_spec(w):
        return pl.BlockSpec((bsz, w), lambda i: (i, 0))

    def full(s):
        return pl.BlockSpec(s, lambda i: (0,) * len(s))

    body = functools.partial(_fused_body, n_rows=bsz, ngrid=ngrid,
                             dim=DIM, k=K, vd=VD, kp=KP)
    xt, ze, lg, idx1, klacc = pl.pallas_call(
        body,
        grid=(ngrid,),
        in_specs=[row_spec(IN), row_spec(8), row_spec(IN),
                  full((IN, DP)), full((1, DP)), full((DP, KP)),
                  full((IN, VD)), full((1, VD)),
                  full((VD, 2 * VD)), full((1, 2 * VD)),
                  full((VD, IN)), full((KP, IN)), full((1, IN))],
        out_specs=[row_spec(IN), row_spec(DIM), row_spec(K),
                   pl.BlockSpec((bsz,), lambda i: (i,)),
                   full((1, 1))],
        out_shape=[jax.ShapeDtypeStruct((B, IN), f32),
                   jax.ShapeDtypeStruct((B, DIM), f32),
                   jax.ShapeDtypeStruct((B, K), f32),
                   jax.ShapeDtypeStruct((B,), jnp.int32),
                   jax.ShapeDtypeStruct((1, 1), f32)],
        compiler_params=pltpu.CompilerParams(
            dimension_semantics=("arbitrary",)),
    )(x, lab8, eps, encWT, encb, embWT, W1T, b1, W2T, b2,
      decWzT, decWyT, decb)

    embp = jnp.zeros((K, DP), f32).at[:, :DIM].set(emb_W)
    z_q = _sc_gather(embp, idx1)[:, :DIM]
    return (xt, ze, z_q, lg, klacc[0, 0])


# speedup vs baseline: 1.1885x; 1.0342x over previous
import functools

import jax
import jax.numpy as jnp
from jax import lax
from jax.experimental import pallas as pl
from jax.experimental.pallas import tpu as pltpu
from jax.experimental.pallas import tpu_sc as plsc

_BSZ = 1024


def _fused_body(x_ref, lab_ref, eps_ref,
                encWT_ref, encb_ref, embWT_ref,
                W1T_ref, b1_ref, W2T_ref, b2_ref,
                decWzT_ref, decWyT_ref, decb_ref,
                xt_ref, ze_ref, lg_ref, idx_ref, kl_ref,
                *, n_rows, ngrid, dim, k, vd, kp):
    i = pl.program_id(0)
    x = x_ref[...]

    ze = jnp.dot(x, encWT_ref[...], preferred_element_type=jnp.float32) + encb_ref[...]
    logits = jnp.dot(ze, embWT_ref[...], preferred_element_type=jnp.float32)

    kiota = lax.broadcasted_iota(jnp.int32, logits.shape, 1)
    lg_m = jnp.where(kiota < k, logits, jnp.float32(-1e30))
    m = jnp.max(lg_m, axis=1, keepdims=True)
    e = jnp.exp(lg_m - m)
    soft = e / jnp.sum(e, axis=1, keepdims=True)
    idx_ref[...] = jnp.min(jnp.where(lg_m >= m, kiota, kp), axis=1)

    sel = lab_ref[...][:, 0:1]
    onehot = (kiota == sel).astype(jnp.float32)
    y = jnp.where(sel >= 0, onehot, soft)

    bf16 = jnp.bfloat16
    h = jnp.maximum(jnp.dot(x.astype(bf16), W1T_ref[...],
                            preferred_element_type=jnp.float32)
                    + b1_ref[...], 0.0)
    mv = jnp.dot(h.astype(bf16), W2T_ref[...],
                 preferred_element_type=jnp.float32) + b2_ref[...]
    mu = mv[:, :vd]
    logvar = mv[:, vd:]
    var = jnp.exp(logvar)
    std = jnp.exp(0.5 * logvar)
    sampled = mu + std * eps_ref[...]

    xt = (jnp.dot(sampled.astype(bf16), decWzT_ref[...],
                  preferred_element_type=jnp.float32)
          + jnp.dot(y.astype(bf16), decWyT_ref[...],
                    preferred_element_type=jnp.float32)
          + decb_ref[...])

    xt_ref[...] = xt
    ze_ref[...] = ze[:, :dim]
    lg_ref[...] = logits[:, :k]

    part = 0.5 * jnp.sum(var + mu * mu - 1.0 - logvar)
    prev = jnp.where(i == 0, jnp.zeros((1, 1), jnp.float32), kl_ref[...])
    tot = prev + part
    kl_ref[...] = jnp.where(i == ngrid - 1, tot / (n_rows * ngrid), tot)


def _sc_gather(table, idx):
    b = idx.shape[0]
    d = table.shape[1]
    nw = 32
    rows_pw = b // nw
    ch = 128
    nch = rows_pw // ch
    mesh = plsc.VectorSubcoreMesh(core_axis_name="c", subcore_axis_name="s")

    @functools.partial(
        pl.kernel, mesh=mesh,
        out_type=jax.ShapeDtypeStruct((b, d), jnp.float32),
        scratch_types=[
            pltpu.VMEM((ch,), jnp.int32),
            pltpu.VMEM((ch, d), jnp.float32),
            pltpu.SemaphoreType.DMA,
        ],
    )
    def k(table_hbm, idx_hbm, out_hbm, idx_v, rows_v, sem):
        wid = lax.axis_index("s") * 2 + lax.axis_index("c")
        base = wid * rows_pw
        for c in range(nch):
            off = base + c * ch
            pltpu.sync_copy(idx_hbm.at[pl.ds(off, ch)], idx_v)
            pltpu.async_copy(table_hbm.at[idx_v], rows_v, sem).wait()
            pltpu.sync_copy(rows_v, out_hbm.at[pl.ds(off, ch)])

    return k(table, idx)


def kernel(x, known_mask, labels, enc_W, enc_b, emb_W,
           vae_enc_W1, vae_enc_b1, vae_enc_W2, vae_enc_b2,
           vae_dec_W, vae_dec_b, eps):
    f32 = jnp.float32
    B, IN = x.shape
    DIM = enc_W.shape[0]
    K = emb_W.shape[0]
    VD = vae_enc_b1.shape[0]
    DP, KP = 512, 128
    bsz = _BSZ
    ngrid = B // bsz

    encWT = jnp.zeros((IN, DP), f32).at[:, :DIM].set(enc_W.T)
    encb = jnp.zeros((1, DP), f32).at[0, :DIM].set(enc_b)
    embWT = jnp.zeros((DP, KP), f32).at[:DIM, :K].set(emb_W.T)
    bf16 = jnp.bfloat16
    W1T = vae_enc_W1.T.astype(bf16)
    b1 = vae_enc_b1[None, :]
    W2T = vae_enc_W2.T.astype(bf16)
    b2 = vae_enc_b2[None, :]
    decWzT = vae_dec_W[:, :VD].T.astype(bf16)
    decWyT = jnp.zeros((KP, IN), bf16).at[:K, :].set(
        vae_dec_W[:, VD:].T.astype(bf16))
    decb = vae_dec_b[None, :]
    labm = jnp.where(known_mask, labels.astype(jnp.int32), -1)
    lab8 = jnp.broadcast_to(labm[:, None], (B, 8))

    def row_spec(w):
        return pl.BlockSpec((bsz, w), lambda i: (i, 0))

    def full(s):
        return pl.BlockSpec(s, lambda i: (0,) * len(s))

    body = functools.partial(_fused_body, n_rows=bsz, ngrid=ngrid,
                             dim=DIM, k=K, vd=VD, kp=KP)
    xt, ze, lg, idx1, klacc = pl.pallas_call(
        body,
        grid=(ngrid,),
        in_specs=[row_spec(IN), row_spec(8), row_spec(IN),
                  full((IN, DP)), full((1, DP)), full((DP, KP)),
                  full((IN, VD)), full((1, VD)),
                  full((VD, 2 * VD)), full((1, 2 * VD)),
                  full((VD, IN)), full((KP, IN)), full((1, IN))],
        out_specs=[row_spec(IN), row_spec(DIM), row_spec(K),
                   pl.BlockSpec((bsz,), lambda i: (i,)),
                   full((1, 1))],
        out_shape=[jax.ShapeDtypeStruct((B, IN), f32),
                   jax.ShapeDtypeStruct((B, DIM), f32),
                   jax.ShapeDtypeStruct((B, K), f32),
                   jax.ShapeDtypeStruct((B,), jnp.int32),
                   jax.ShapeDtypeStruct((1, 1), f32)],
        compiler_params=pltpu.CompilerParams(
            dimension_semantics=("arbitrary",)),
    )(x, lab8, eps, encWT, encb, embWT, W1T, b1, W2T, b2,
      decWzT, decWyT, decb)

    embp = jnp.zeros((K, DP), f32).at[:, :DIM].set(emb_W)
    z_q = _sc_gather(embp, idx1)[:, :DIM]
    return (xt, ze, z_q, lg, klacc[0, 0])
